# baseline (speedup 1.0000x reference)
import dataclasses

import jax
import jax.numpy as jnp
from jax import lax
from jax.experimental import pallas as pl
from jax.experimental.pallas import tpu as pltpu
from jax.experimental.pallas import tpu_sc as plsc

_N = 786432
_C = 91
_L = 16
_NW = 32
_ROWS_W = _N // _NW
_CH = 2048
_NCH = _ROWS_W // _CH
_GPC = _CH // _L

_W = 65536
_KSUB = _W // 128
_NB128 = _N // 128

_N_TC = 655360
_N_SCE = _N - _N_TC
_SCE_W = _N_SCE // _NW
_CCH = 256
_SCE_NCH = _SCE_W // _CCH
_SCE_GPC = _CCH // _L

_LN2 = 0.6931471805599453
_SQRT2 = 1.4142135623730951


def _vlog(x):
    bits = plsc.bitcast(x, jnp.int32)
    e = lax.shift_right_logical(bits, 23) - 127
    m = plsc.bitcast((bits & 0x007FFFFF) | 0x3F800000, jnp.float32)
    big = m > _SQRT2
    m = jnp.where(big, m * 0.5, m)
    ef = e.astype(jnp.float32) + jnp.where(big, 1.0, 0.0)
    t = (m - 1.0) / (m + 1.0)
    t2 = t * t
    p = 2.0 + t2 * (2.0 / 3.0 + t2 * (2.0 / 5.0 + t2 * (2.0 / 7.0 + t2 * (2.0 / 9.0))))
    return ef * _LN2 + t * p


def _tc_body(cls_ref, lab_ref, gobj_ref, out_ref):
    @pl.when(pl.program_id(0) == 0)
    def _():
        out_ref[...] = jnp.zeros_like(out_ref)

    iot = lax.broadcasted_iota(jnp.int32, (_C, 128), 0)
    acc_ce = jnp.zeros((1, 128), jnp.float32)
    acc_nb = jnp.zeros((1, 128), jnp.float32)
    for k in range(_KSUB):
        exs = jnp.exp(cls_ref[:, 128 * k:128 * (k + 1)])
        lab = jnp.clip(lab_ref[k:k + 1, :], 0, _C - 1)
        gob = gobj_ref[k:k + 1, :]
        sexp = jnp.sum(exs, axis=0, keepdims=True)
        explab = jnp.sum(jnp.where(iot == lab, exs, 0.0), axis=0,
                         keepdims=True)
        ce = jnp.log(sexp / explab)
        mbb = jnp.where(gob == 1, 1.0, 0.0).astype(jnp.float32)
        acc_ce = acc_ce + ce * mbb
        acc_nb = acc_nb + mbb
    out_ref[0:1, :] += acc_ce
    out_ref[1:2, :] += acc_nb


@jax.jit
def _tc_ce(cls_t, lab2d, gobj2d):
    return pl.pallas_call(
        _tc_body,
        grid=(_N_TC // _W,),
        in_specs=[
            pl.BlockSpec((_C, _W), lambda i: (0, i)),
            pl.BlockSpec((_KSUB, 128), lambda i: (i, 0)),
            pl.BlockSpec((_KSUB, 128), lambda i: (i, 0)),
        ],
        out_specs=pl.BlockSpec((2, 128), lambda i: (0, 0)),
        out_shape=jax.ShapeDtypeStruct((2, 128), jnp.float32),
    )(cls_t, lab2d, gobj2d)


def _sc_body(obj_hbm, off_hbm, goff_hbm, gobj_hbm, cls_hbm, gcls_hbm,
             out_hbm, obj_v, off_v, goff_v, gobj_v, cls_v, gcls_v, gobj2_v,
             acc_v, sem):
    cid = lax.axis_index("c")
    sid = lax.axis_index("s")
    wid = sid * 2 + cid
    base = wid * _ROWS_W

    def _copies(ci, b):
        a0 = base + ci * _CH
        cps = []
        for r in range(2):
            cps.append(pltpu.make_async_copy(
                obj_hbm.at[r, pl.ds(a0, _CH)], obj_v.at[b, r], sem.at[b]))
        for r in range(4):
            cps.append(pltpu.make_async_copy(
                off_hbm.at[r, pl.ds(a0, _CH)], off_v.at[b, r], sem.at[b]))
            cps.append(pltpu.make_async_copy(
                goff_hbm.at[r, pl.ds(a0, _CH)], goff_v.at[b, r], sem.at[b]))
        cps.append(pltpu.make_async_copy(
            gobj_hbm.at[pl.ds(a0, _CH)], gobj_v.at[b], sem.at[b]))
        return cps

    def group_body_for(b):
        def group_body(g, carry):
            focal_a, sl1_a, nobj_a = carry
            sl = pl.ds(g * _L, _L)
            gobj = gobj_v[b, sl]
            m_obj = jnp.where(gobj != -1, 1.0, 0.0).astype(jnp.float32)
            m_bb = jnp.where(gobj == 1, 1.0, 0.0).astype(jnp.float32)

            a = obj_v[b, 0, sl]
            bb = obj_v[b, 1, sl]
            ea = jnp.exp(a)
            eb = jnp.exp(bb)
            s2 = ea + eb
            pos = gobj >= 1
            xl2 = jnp.where(pos, bb, a)
            el2 = jnp.where(pos, eb, ea)
            logpt = xl2 - _vlog(s2)
            pt = el2 / s2
            q = 1.0 - pt
            focal = -(q * q) * logpt

            sl1 = jnp.zeros((_L,), jnp.float32)
            for c in range(4):
                d = off_v[b, c, sl] - goff_v[b, c, sl]
                ad = jnp.abs(d)
                sl1 = sl1 + jnp.where(ad < 1.0, 0.5 * ad * ad, ad - 0.5)

            return (focal_a + focal * m_obj, sl1_a + sl1 * m_bb,
                    nobj_a + m_obj)
        return group_body

    for cp in _copies(0, 0):
        cp.start()

    def pair_body(p, carry):
        for b in range(2):
            ci = 2 * p + b
            nxt_ok = ci + 1 < _NCH

            @pl.when(nxt_ok)
            def _():
                for cp in _copies(ci + 1, 1 - b):
                    cp.start()

            for cp in _copies(ci, b):
                cp.wait()
            carry = lax.fori_loop(0, _GPC, group_body_for(b), carry)
        return carry

    z = jnp.zeros((_L,), jnp.float32)
    focal_a, sl1_a, nobj_a = lax.fori_loop(0, _NCH // 2, pair_body, (z, z, z))

    cbase = _N_TC + wid * _SCE_W
    lane = lax.iota(jnp.int32, _L)

    def ce_chunk(ci, carry):
        a0 = cbase + ci * _CCH
        pltpu.sync_copy(cls_hbm.at[:, pl.ds(a0, _CCH)], cls_v)
        pltpu.sync_copy(gcls_hbm.at[pl.ds(a0, _CCH)], gcls_v)
        pltpu.sync_copy(gobj_hbm.at[pl.ds(a0, _CCH)], gobj2_v)

        def ce_group(g, carry2):
            ce_a, nbb_a = carry2
            sl = pl.ds(g * _L, _L)
            zz = jnp.zeros((_L,), jnp.float32)
            acc4 = [zz, zz, zz, zz]
            for c in range(_C):
                acc4[c % 4] = acc4[c % 4] + jnp.exp(cls_v[c, sl])
            sexp = (acc4[0] + acc4[1]) + (acc4[2] + acc4[3])
            lbl = jnp.clip(gcls_v[sl], 0, _C - 1)
            cols = g * _L + lane
            xlab = plsc.load_gather(cls_v, [lbl, cols])
            m_bb = jnp.where(gobj2_v[sl] == 1, 1.0, 0.0).astype(jnp.float32)
            ce = _vlog(sexp) - xlab
            return (ce_a + ce * m_bb, nbb_a + m_bb)

        return lax.fori_loop(0, _SCE_GPC, ce_group, carry)

    ce_a, nbb_a = lax.fori_loop(0, _SCE_NCH, ce_chunk, (z, z))

    acc_v[pl.ds(0, _L)] = focal_a
    acc_v[pl.ds(_L, _L)] = sl1_a
    acc_v[pl.ds(2 * _L, _L)] = nobj_a
    acc_v[pl.ds(3 * _L, _L)] = ce_a
    acc_v[pl.ds(4 * _L, _L)] = nbb_a
    pltpu.sync_copy(acc_v, out_hbm.at[pl.ds(wid * 5 * _L, 5 * _L)])


@jax.jit
def _sc_partials(obj_t, off_t, goff_t, gobj, cls_t, gcls):
    cp = pltpu.CompilerParams()
    if "needs_layout_passes" in pltpu.CompilerParams.__dataclass_fields__:
        cp = dataclasses.replace(cp, needs_layout_passes=False)
    mesh = plsc.VectorSubcoreMesh(core_axis_name="c", subcore_axis_name="s")
    run = pl.kernel(
        _sc_body,
        out_type=jax.ShapeDtypeStruct((_NW * 5 * _L,), jnp.float32),
        mesh=mesh,
        scratch_types=[
            pltpu.VMEM((2, 2, _CH), jnp.float32),
            pltpu.VMEM((2, 4, _CH), jnp.float32),
            pltpu.VMEM((2, 4, _CH), jnp.float32),
            pltpu.VMEM((2, _CH), jnp.int32),
            pltpu.VMEM((_C, _CCH), jnp.float32),
            pltpu.VMEM((_CCH,), jnp.int32),
            pltpu.VMEM((_CCH,), jnp.int32),
            pltpu.VMEM((5 * _L,), jnp.float32),
            pltpu.SemaphoreType.DMA((2,)),
        ],
        compiler_params=cp,
    )
    return run(obj_t, off_t, goff_t, gobj, cls_t, gcls)


def kernel(bb_targets_offset, bb_targets_cls, bb_targets_objectness,
           gt_bb_targets_offset, s_obj, s_cls, s_bb, gt_bb_targets_cls,
           gt_bb_targets_objectness, step):
    cls_t = jnp.reshape(bb_targets_cls, (_N, _C)).T
    obj_t = jnp.reshape(bb_targets_objectness, (_N, 2)).T
    off_t = jnp.reshape(bb_targets_offset, (_N, 4)).T
    goff_t = jnp.reshape(gt_bb_targets_offset, (_N, 4)).T
    gobj = jnp.reshape(gt_bb_targets_objectness, (_N,))
    gcls = jnp.reshape(gt_bb_targets_cls, (_N,))
    lab2d = jnp.reshape(gcls, (_NB128, 128))
    gobj2d = jnp.reshape(gobj, (_NB128, 128))

    tc = _tc_ce(cls_t, lab2d, gobj2d)
    sc = jnp.reshape(_sc_partials(obj_t, off_t, goff_t, gobj, cls_t, gcls),
                     (_NW, 5, _L))

    p = jnp.sum(sc, axis=(0, 2))
    focal_s, sl1_s, n_obj = p[0], p[1], p[2]
    ce_s = jnp.sum(tc[0]) + p[3]
    n_bb = jnp.sum(tc[1]) + p[4]

    obj_loss = jnp.where(n_obj > 0, focal_s / jnp.maximum(n_obj, 1.0), 0.0) * 0.1
    cls_loss = jnp.where(n_bb > 0, ce_s / jnp.maximum(n_bb, 1.0), 0.0) * 50.0
    bb_loss = jnp.where(n_bb > 0, sl1_s / (4.0 * jnp.maximum(n_bb, 1.0)), 0.0) * 100.0

    obj_loss = obj_loss * jnp.exp(-s_obj) + s_obj
    cls_loss = cls_loss * jnp.exp(-s_cls) + s_cls
    bb_loss = bb_loss * jnp.exp(-s_bb) + s_bb
    return (cls_loss, obj_loss, bb_loss)


# speedup vs baseline: 1.0799x; 1.0799x over previous
import dataclasses

import jax
import jax.numpy as jnp
from jax import lax
from jax.experimental import pallas as pl
from jax.experimental.pallas import tpu as pltpu
from jax.experimental.pallas import tpu_sc as plsc

_N = 786432
_C = 91
_L = 16
_NW = 32
_ROWS_W = _N // _NW
_CH = 2048
_NCH = _ROWS_W // _CH
_GPC = _CH // _L

_W = 65536
_KSUB = _W // 128
_NB128 = _N // 128

_N_TC = 720896
_N_SCE = _N - _N_TC
_SCE_W = _N_SCE // _NW
_CCH = 256
_SCE_NCH = _SCE_W // _CCH
_SCE_GPC = _CCH // _L

_LN2 = 0.6931471805599453
_SQRT2 = 1.4142135623730951


def _vlog(x):
    bits = plsc.bitcast(x, jnp.int32)
    e = lax.shift_right_logical(bits, 23) - 127
    m = plsc.bitcast((bits & 0x007FFFFF) | 0x3F800000, jnp.float32)
    big = m > _SQRT2
    m = jnp.where(big, m * 0.5, m)
    ef = e.astype(jnp.float32) + jnp.where(big, 1.0, 0.0)
    t = (m - 1.0) / (m + 1.0)
    t2 = t * t
    p = 2.0 + t2 * (2.0 / 3.0 + t2 * (2.0 / 5.0 + t2 * (2.0 / 7.0 + t2 * (2.0 / 9.0))))
    return ef * _LN2 + t * p


def _tc_body(cls_ref, lab_ref, gobj_ref, out_ref):
    @pl.when(pl.program_id(0) == 0)
    def _():
        out_ref[...] = jnp.zeros_like(out_ref)

    iot = lax.broadcasted_iota(jnp.int32, (_C, 128), 0)
    acc_ce = jnp.zeros((1, 128), jnp.float32)
    acc_nb = jnp.zeros((1, 128), jnp.float32)
    for k in range(_KSUB):
        exs = jnp.exp(cls_ref[:, 128 * k:128 * (k + 1)])
        lab = jnp.clip(lab_ref[k:k + 1, :], 0, _C - 1)
        gob = gobj_ref[k:k + 1, :]
        sexp = jnp.sum(exs, axis=0, keepdims=True)
        explab = jnp.sum(jnp.where(iot == lab, exs, 0.0), axis=0,
                         keepdims=True)
        ce = jnp.log(sexp / explab)
        mbb = jnp.where(gob == 1, 1.0, 0.0).astype(jnp.float32)
        acc_ce = acc_ce + ce * mbb
        acc_nb = acc_nb + mbb
    out_ref[0:1, :] += acc_ce
    out_ref[1:2, :] += acc_nb


@jax.jit
def _tc_ce(cls_t, lab2d, gobj2d):
    return pl.pallas_call(
        _tc_body,
        grid=(_N_TC // _W,),
        in_specs=[
            pl.BlockSpec((_C, _W), lambda i: (0, i)),
            pl.BlockSpec((_KSUB, 128), lambda i: (i, 0)),
            pl.BlockSpec((_KSUB, 128), lambda i: (i, 0)),
        ],
        out_specs=pl.BlockSpec((2, 128), lambda i: (0, 0)),
        out_shape=jax.ShapeDtypeStruct((2, 128), jnp.float32),
    )(cls_t, lab2d, gobj2d)


def _sc_body(obj_hbm, off_hbm, goff_hbm, gobj_hbm, cls_hbm, gcls_hbm,
             out_hbm, obj_v, off_v, goff_v, gobj_v, cls_v, gcls_v, gobj2_v,
             acc_v, sem):
    cid = lax.axis_index("c")
    sid = lax.axis_index("s")
    wid = sid * 2 + cid
    base = wid * _ROWS_W

    def _copies(ci, b):
        a0 = base + ci * _CH
        cps = []
        for r in range(2):
            cps.append(pltpu.make_async_copy(
                obj_hbm.at[r, pl.ds(a0, _CH)], obj_v.at[b, r], sem.at[b]))
        for r in range(4):
            cps.append(pltpu.make_async_copy(
                off_hbm.at[r, pl.ds(a0, _CH)], off_v.at[b, r], sem.at[b]))
            cps.append(pltpu.make_async_copy(
                goff_hbm.at[r, pl.ds(a0, _CH)], goff_v.at[b, r], sem.at[b]))
        cps.append(pltpu.make_async_copy(
            gobj_hbm.at[pl.ds(a0, _CH)], gobj_v.at[b], sem.at[b]))
        return cps

    def group_body_for(b):
        def group_body(g, carry):
            focal_a, sl1_a, nobj_a = carry
            sl = pl.ds(g * _L, _L)
            gobj = gobj_v[b, sl]
            m_obj = jnp.where(gobj != -1, 1.0, 0.0).astype(jnp.float32)
            m_bb = jnp.where(gobj == 1, 1.0, 0.0).astype(jnp.float32)

            a = obj_v[b, 0, sl]
            bb = obj_v[b, 1, sl]
            ea = jnp.exp(a)
            eb = jnp.exp(bb)
            s2 = ea + eb
            pos = gobj >= 1
            xl2 = jnp.where(pos, bb, a)
            el2 = jnp.where(pos, eb, ea)
            logpt = xl2 - _vlog(s2)
            pt = el2 / s2
            q = 1.0 - pt
            focal = -(q * q) * logpt

            sl1 = jnp.zeros((_L,), jnp.float32)
            for c in range(4):
                d = off_v[b, c, sl] - goff_v[b, c, sl]
                ad = jnp.abs(d)
                sl1 = sl1 + jnp.where(ad < 1.0, 0.5 * ad * ad, ad - 0.5)

            return (focal_a + focal * m_obj, sl1_a + sl1 * m_bb,
                    nobj_a + m_obj)
        return group_body

    for cp in _copies(0, 0):
        cp.start()

    def pair_body(p, carry):
        for b in range(2):
            ci = 2 * p + b
            nxt_ok = ci + 1 < _NCH

            @pl.when(nxt_ok)
            def _():
                for cp in _copies(ci + 1, 1 - b):
                    cp.start()

            for cp in _copies(ci, b):
                cp.wait()
            carry = lax.fori_loop(0, _GPC, group_body_for(b), carry)
        return carry

    z = jnp.zeros((_L,), jnp.float32)
    focal_a, sl1_a, nobj_a = lax.fori_loop(0, _NCH // 2, pair_body, (z, z, z))

    cbase = _N_TC + wid * _SCE_W
    lane = lax.iota(jnp.int32, _L)

    def ce_chunk(ci, carry):
        a0 = cbase + ci * _CCH
        pltpu.sync_copy(cls_hbm.at[:, pl.ds(a0, _CCH)], cls_v)
        pltpu.sync_copy(gcls_hbm.at[pl.ds(a0, _CCH)], gcls_v)
        pltpu.sync_copy(gobj_hbm.at[pl.ds(a0, _CCH)], gobj2_v)

        def ce_group(g, carry2):
            ce_a, nbb_a = carry2
            sl = pl.ds(g * _L, _L)
            zz = jnp.zeros((_L,), jnp.float32)
            acc4 = [zz, zz, zz, zz]
            for c in range(_C):
                acc4[c % 4] = acc4[c % 4] + jnp.exp(cls_v[c, sl])
            sexp = (acc4[0] + acc4[1]) + (acc4[2] + acc4[3])
            lbl = jnp.clip(gcls_v[sl], 0, _C - 1)
            cols = g * _L + lane
            xlab = plsc.load_gather(cls_v, [lbl, cols])
            m_bb = jnp.where(gobj2_v[sl] == 1, 1.0, 0.0).astype(jnp.float32)
            ce = _vlog(sexp) - xlab
            return (ce_a + ce * m_bb, nbb_a + m_bb)

        return lax.fori_loop(0, _SCE_GPC, ce_group, carry)

    ce_a, nbb_a = lax.fori_loop(0, _SCE_NCH, ce_chunk, (z, z))

    acc_v[pl.ds(0, _L)] = focal_a
    acc_v[pl.ds(_L, _L)] = sl1_a
    acc_v[pl.ds(2 * _L, _L)] = nobj_a
    acc_v[pl.ds(3 * _L, _L)] = ce_a
    acc_v[pl.ds(4 * _L, _L)] = nbb_a
    pltpu.sync_copy(acc_v, out_hbm.at[pl.ds(wid * 5 * _L, 5 * _L)])


@jax.jit
def _sc_partials(obj_t, off_t, goff_t, gobj, cls_t, gcls):
    cp = pltpu.CompilerParams()
    if "needs_layout_passes" in pltpu.CompilerParams.__dataclass_fields__:
        cp = dataclasses.replace(cp, needs_layout_passes=False)
    mesh = plsc.VectorSubcoreMesh(core_axis_name="c", subcore_axis_name="s")
    run = pl.kernel(
        _sc_body,
        out_type=jax.ShapeDtypeStruct((_NW * 5 * _L,), jnp.float32),
        mesh=mesh,
        scratch_types=[
            pltpu.VMEM((2, 2, _CH), jnp.float32),
            pltpu.VMEM((2, 4, _CH), jnp.float32),
            pltpu.VMEM((2, 4, _CH), jnp.float32),
            pltpu.VMEM((2, _CH), jnp.int32),
            pltpu.VMEM((_C, _CCH), jnp.float32),
            pltpu.VMEM((_CCH,), jnp.int32),
            pltpu.VMEM((_CCH,), jnp.int32),
            pltpu.VMEM((5 * _L,), jnp.float32),
            pltpu.SemaphoreType.DMA((2,)),
        ],
        compiler_params=cp,
    )
    return run(obj_t, off_t, goff_t, gobj, cls_t, gcls)


def kernel(bb_targets_offset, bb_targets_cls, bb_targets_objectness,
           gt_bb_targets_offset, s_obj, s_cls, s_bb, gt_bb_targets_cls,
           gt_bb_targets_objectness, step):
    cls_t = jnp.reshape(bb_targets_cls, (_N, _C)).T
    obj_t = jnp.reshape(bb_targets_objectness, (_N, 2)).T
    off_t = jnp.reshape(bb_targets_offset, (_N, 4)).T
    goff_t = jnp.reshape(gt_bb_targets_offset, (_N, 4)).T
    gobj = jnp.reshape(gt_bb_targets_objectness, (_N,))
    gcls = jnp.reshape(gt_bb_targets_cls, (_N,))
    lab2d = jnp.reshape(gcls, (_NB128, 128))
    gobj2d = jnp.reshape(gobj, (_NB128, 128))

    tc = _tc_ce(cls_t, lab2d, gobj2d)
    sc = jnp.reshape(_sc_partials(obj_t, off_t, goff_t, gobj, cls_t, gcls),
                     (_NW, 5, _L))

    p = jnp.sum(sc, axis=(0, 2))
    focal_s, sl1_s, n_obj = p[0], p[1], p[2]
    ce_s = jnp.sum(tc[0]) + p[3]
    n_bb = jnp.sum(tc[1]) + p[4]

    obj_loss = jnp.where(n_obj > 0, focal_s / jnp.maximum(n_obj, 1.0), 0.0) * 0.1
    cls_loss = jnp.where(n_bb > 0, ce_s / jnp.maximum(n_bb, 1.0), 0.0) * 50.0
    bb_loss = jnp.where(n_bb > 0, sl1_s / (4.0 * jnp.maximum(n_bb, 1.0)), 0.0) * 100.0

    obj_loss = obj_loss * jnp.exp(-s_obj) + s_obj
    cls_loss = cls_loss * jnp.exp(-s_cls) + s_cls
    bb_loss = bb_loss * jnp.exp(-s_bb) + s_bb
    return (cls_loss, obj_loss, bb_loss)
